# baseline (speedup 1.0000x reference)
import functools

import jax
import jax.numpy as jnp
from jax import lax
from jax.experimental import pallas as pl
from jax.experimental.pallas import tpu as pltpu

_H = 4


def _attention_layer(h, W_ref, al_ref, ar_ref, b_ref, maskadd, ones_col,
                     Fo, act, mean_heads):
    feat = jnp.dot(h.astype(jnp.bfloat16), W_ref[...],
                   preferred_element_type=jnp.float32)
    outs = None
    for hd in range(_H):
        f_h = feat[:, hd * Fo:(hd + 1) * Fo]
        al_h = al_ref[hd:hd + 1, :]
        ar_h = ar_ref[hd:hd + 1, :]
        er = jnp.sum(f_h * ar_h, axis=1, keepdims=True)
        el = lax.dot_general(al_h, f_h, (((1,), (1,)), ((), ())),
                             preferred_element_type=jnp.float32)
        e = er + el
        e = jnp.maximum(e, 0.2 * e) + maskadd
        emax = jnp.max(e, axis=1, keepdims=True)
        ee = jnp.exp(e - emax)
        denom = jnp.sum(ee, axis=1, keepdims=True)
        alpha = ee * (1.0 / jnp.maximum(denom, 1e-9))
        o_h = lax.dot_general(alpha, f_h, (((1,), (0,)), ((), ())),
                              preferred_element_type=jnp.float32)
        o_h = o_h + b_ref[:, hd * Fo:(hd + 1) * Fo]
        if mean_heads:
            outs = o_h if outs is None else outs + o_h
        else:
            outs = o_h if outs is None else jnp.concatenate([outs, o_h], axis=1)
    if mean_heads:
        outs = outs * (1.0 / _H)
    if act:
        outs = jnp.maximum(outs, 0.0)
    return outs


def _gat_kernel(node_nums_ref, x_ref, adj_ref,
                W0_ref, al0_ref, ar0_ref, b0_ref,
                W1_ref, al1_ref, ar1_ref, b1_ref,
                W2_ref, al2_ref, ar2_ref, b2_ref,
                out_ref, *, Nmax, Fh, Fout):
    b = pl.program_id(0)
    nn = jnp.maximum(node_nums_ref[b], 1)
    ii = lax.broadcasted_iota(jnp.int32, (Nmax, Nmax), 0)
    jj = lax.broadcasted_iota(jnp.int32, (Nmax, Nmax), 1)
    mask = (adj_ref[0, 0] != 0) & (ii < nn) & (jj < nn)
    maskadd = jnp.where(mask, 0.0, -1e30).T
    ones_col = jnp.ones((Nmax, 1), jnp.float32)

    h = x_ref[0, 0]
    h = _attention_layer(h, W0_ref, al0_ref, ar0_ref, b0_ref, maskadd,
                         ones_col, Fh, act=True, mean_heads=False)
    h = _attention_layer(h, W1_ref, al1_ref, ar1_ref, b1_ref, maskadd,
                         ones_col, Fh, act=True, mean_heads=False)
    h = _attention_layer(h, W2_ref, al2_ref, ar2_ref, b2_ref, maskadd,
                         ones_col, Fout, act=False, mean_heads=True)
    valid_col = lax.broadcasted_iota(jnp.int32, (Nmax, 1), 0) < nn
    out_ref[0] = jnp.where(valid_col, h, 0.0)


def kernel(x, adj, node_nums, W0, al0, ar0, b0, W1, al1, ar1, b1,
           W2, al2, ar2, b2):
    B, C, Nmax, Fin = x.shape
    Hh, Fh = al0.shape
    Fout = al2.shape[1]
    HF = Hh * Fh

    b0r = b0.reshape(1, HF)
    b1r = b1.reshape(1, HF)
    b2r = b2.reshape(1, Hh * Fout)
    x = x.astype(jnp.bfloat16)
    W0 = W0.astype(jnp.bfloat16)
    W1 = W1.astype(jnp.bfloat16)
    W2 = W2.astype(jnp.bfloat16)

    def full(shape):
        return pl.BlockSpec(shape, lambda b, *_: (0,) * len(shape))

    grid_spec = pltpu.PrefetchScalarGridSpec(
        num_scalar_prefetch=1,
        grid=(B,),
        in_specs=[
            pl.BlockSpec((1, 1, Nmax, Fin), lambda b, *_: (b, 0, 0, 0)),
            pl.BlockSpec((1, 1, Nmax, Nmax), lambda b, *_: (b, 0, 0, 0)),
            full((Fin, HF)), full((Hh, Fh)), full((Hh, Fh)), full((1, HF)),
            full((HF, HF)), full((Hh, Fh)), full((Hh, Fh)), full((1, HF)),
            full((HF, Hh * Fout)), full((Hh, Fout)), full((Hh, Fout)),
            full((1, Hh * Fout)),
        ],
        out_specs=pl.BlockSpec((1, Nmax, Fout), lambda b, *_: (b, 0, 0)),
    )

    out = pl.pallas_call(
        functools.partial(_gat_kernel, Nmax=Nmax, Fh=Fh, Fout=Fout),
        grid_spec=grid_spec,
        out_shape=jax.ShapeDtypeStruct((B, Nmax, Fout), jnp.float32),
        compiler_params=pltpu.CompilerParams(
            dimension_semantics=("parallel",)),
    )(node_nums.astype(jnp.int32), x, adj,
      W0, al0, ar0, b0r, W1, al1, ar1, b1r, W2, al2, ar2, b2r)
    return out


# speedup vs baseline: 1.5481x; 1.5481x over previous
import functools

import jax
import jax.numpy as jnp
from jax import lax
from jax.experimental import pallas as pl
from jax.experimental.pallas import tpu as pltpu

_H = 4


def _attention_layer(h, W_ref, al_ref, ar_ref, b_ref, mask01, ones_col,
                     Fo, act, mean_heads):
    feat = jnp.dot(h, W_ref[...], preferred_element_type=jnp.float32)
    outs = None
    for hd in range(_H):
        f_h = feat[:, hd * Fo:(hd + 1) * Fo]
        al_h = al_ref[hd:hd + 1, :]
        ar_h = ar_ref[hd:hd + 1, :]
        er = jnp.sum(f_h * ar_h, axis=1, keepdims=True)
        el = lax.dot_general(al_h, f_h, (((1,), (1,)), ((), ())),
                             preferred_element_type=jnp.float32)
        elmax = jnp.max(el)
        ermax = jnp.max(er)
        a_row = jnp.exp(el - elmax)
        u_row = jnp.exp(0.2 * el - elmax)
        b_col = jnp.exp(er - ermax)
        v_col = jnp.exp(0.2 * er - ermax)
        ee = jnp.maximum(b_col * a_row, v_col * u_row) * mask01
        denom = jnp.sum(ee, axis=1, keepdims=True)
        o_h = lax.dot_general(ee, f_h, (((1,), (0,)), ((), ())),
                              preferred_element_type=jnp.float32)
        o_h = o_h * (1.0 / jnp.maximum(denom, 1e-9))
        o_h = o_h + b_ref[:, hd * Fo:(hd + 1) * Fo]
        if mean_heads:
            outs = o_h if outs is None else outs + o_h
        else:
            outs = o_h if outs is None else jnp.concatenate([outs, o_h], axis=1)
    if mean_heads:
        outs = outs * (1.0 / _H)
    if act:
        outs = jnp.maximum(outs, 0.0)
    return outs


def _gat_kernel(node_nums_ref, x_ref, adj_ref,
                W0_ref, al0_ref, ar0_ref, b0_ref,
                W1_ref, al1_ref, ar1_ref, b1_ref,
                W2_ref, al2_ref, ar2_ref, b2_ref,
                out_ref, *, Nmax, Fh, Fout):
    b = pl.program_id(0)
    nn = jnp.maximum(node_nums_ref[b], 1)
    ii = lax.broadcasted_iota(jnp.int32, (Nmax, Nmax), 0)
    jj = lax.broadcasted_iota(jnp.int32, (Nmax, Nmax), 1)
    mask = (adj_ref[0, 0] != 0) & (ii < nn) & (jj < nn)
    mask01 = jnp.where(mask, 1.0, 0.0).T
    ones_col = jnp.ones((Nmax, 1), jnp.float32)

    h = x_ref[0, 0]
    h = _attention_layer(h, W0_ref, al0_ref, ar0_ref, b0_ref, mask01,
                         ones_col, Fh, act=True, mean_heads=False)
    h = _attention_layer(h, W1_ref, al1_ref, ar1_ref, b1_ref, mask01,
                         ones_col, Fh, act=True, mean_heads=False)
    h = _attention_layer(h, W2_ref, al2_ref, ar2_ref, b2_ref, mask01,
                         ones_col, Fout, act=False, mean_heads=True)
    valid_col = lax.broadcasted_iota(jnp.int32, (Nmax, 1), 0) < nn
    out_ref[0] = jnp.where(valid_col, h, 0.0)


def kernel(x, adj, node_nums, W0, al0, ar0, b0, W1, al1, ar1, b1,
           W2, al2, ar2, b2):
    B, C, Nmax, Fin = x.shape
    Hh, Fh = al0.shape
    Fout = al2.shape[1]
    HF = Hh * Fh

    b0r = b0.reshape(1, HF)
    b1r = b1.reshape(1, HF)
    b2r = b2.reshape(1, Hh * Fout)

    def full(shape):
        return pl.BlockSpec(shape, lambda b, *_: (0,) * len(shape))

    grid_spec = pltpu.PrefetchScalarGridSpec(
        num_scalar_prefetch=1,
        grid=(B,),
        in_specs=[
            pl.BlockSpec((1, 1, Nmax, Fin), lambda b, *_: (b, 0, 0, 0)),
            pl.BlockSpec((1, 1, Nmax, Nmax), lambda b, *_: (b, 0, 0, 0)),
            full((Fin, HF)), full((Hh, Fh)), full((Hh, Fh)), full((1, HF)),
            full((HF, HF)), full((Hh, Fh)), full((Hh, Fh)), full((1, HF)),
            full((HF, Hh * Fout)), full((Hh, Fout)), full((Hh, Fout)),
            full((1, Hh * Fout)),
        ],
        out_specs=pl.BlockSpec((1, Nmax, Fout), lambda b, *_: (b, 0, 0)),
    )

    out = pl.pallas_call(
        functools.partial(_gat_kernel, Nmax=Nmax, Fh=Fh, Fout=Fout),
        grid_spec=grid_spec,
        out_shape=jax.ShapeDtypeStruct((B, Nmax, Fout), jnp.float32),
        compiler_params=pltpu.CompilerParams(
            dimension_semantics=("parallel",)),
    )(node_nums.astype(jnp.int32), x, adj,
      W0, al0, ar0, b0r, W1, al1, ar1, b1r, W2, al2, ar2, b2r)
    return out


# speedup vs baseline: 1.6295x; 1.0525x over previous
import functools

import jax
import jax.numpy as jnp
from jax import lax
from jax.experimental import pallas as pl
from jax.experimental.pallas import tpu as pltpu

_H = 4


def _attention_layer(h_all, W_ref, al_ref, ar_ref, b_ref, masks,
                     Fo, act, mean_heads, Nmax):
    feat = jnp.dot(h_all, W_ref[...],
                   preferred_element_type=jnp.float32)
    g_outs = []
    for g, mask01 in enumerate(masks):
        outs = None
        for hd in range(_H):
            f_h = feat[g * Nmax:(g + 1) * Nmax, hd * Fo:(hd + 1) * Fo]
            al_h = al_ref[hd:hd + 1, :]
            ar_h = ar_ref[hd:hd + 1, :]
            er = lax.dot_general(f_h, ar_h, (((1,), (1,)), ((), ())),
                                 preferred_element_type=jnp.float32)
            el = lax.dot_general(al_h, f_h, (((1,), (1,)), ((), ())),
                                 preferred_element_type=jnp.float32)
            elmax = jnp.max(el)
            ermax = jnp.max(er)
            a_row = jnp.exp(el - elmax)
            u_row = jnp.exp(0.2 * el - elmax)
            b_col = jnp.exp(er - ermax)
            v_col = jnp.exp(0.2 * er - ermax)
            ee = jnp.maximum(b_col * a_row, v_col * u_row) * mask01
            denom = jnp.sum(ee, axis=1, keepdims=True)
            o_h = lax.dot_general(ee, f_h, (((1,), (0,)), ((), ())),
                                  preferred_element_type=jnp.float32)
            o_h = o_h * (1.0 / jnp.maximum(denom, 1e-9))
            o_h = o_h + b_ref[:, hd * Fo:(hd + 1) * Fo]
            if mean_heads:
                outs = o_h if outs is None else outs + o_h
            else:
                outs = o_h if outs is None else jnp.concatenate(
                    [outs, o_h], axis=1)
        if mean_heads:
            outs = outs * (1.0 / _H)
        if act:
            outs = jnp.maximum(outs, 0.0)
        g_outs.append(outs)
    return jnp.concatenate(g_outs, axis=0)


def _gat_kernel(node_nums_ref, x_ref, adj_ref,
                W0_ref, al0_ref, ar0_ref, b0_ref,
                W1_ref, al1_ref, ar1_ref, b1_ref,
                W2_ref, al2_ref, ar2_ref, b2_ref,
                out_ref, *, B, Nmax, Fin, Fh, Fout):
    ii = lax.broadcasted_iota(jnp.int32, (Nmax, Nmax), 0)
    jj = lax.broadcasted_iota(jnp.int32, (Nmax, Nmax), 1)
    masks = []
    for g in range(B):
        nn = jnp.maximum(node_nums_ref[g], 1)
        mask = (adj_ref[g, 0] != 0) & (ii < nn) & (jj < nn)
        masks.append(jnp.where(mask, 1.0, 0.0).T)

    h = x_ref[...].reshape(B * Nmax, Fin)
    h = _attention_layer(h, W0_ref, al0_ref, ar0_ref, b0_ref, masks,
                         Fh, act=True, mean_heads=False, Nmax=Nmax)
    h = _attention_layer(h, W1_ref, al1_ref, ar1_ref, b1_ref, masks,
                         Fh, act=True, mean_heads=False, Nmax=Nmax)
    h = _attention_layer(h, W2_ref, al2_ref, ar2_ref, b2_ref, masks,
                         Fout, act=False, mean_heads=True, Nmax=Nmax)
    for g in range(B):
        nn = jnp.maximum(node_nums_ref[g], 1)
        valid_col = lax.broadcasted_iota(jnp.int32, (Nmax, 1), 0) < nn
        out_ref[g] = jnp.where(valid_col, h[g * Nmax:(g + 1) * Nmax], 0.0)


def kernel(x, adj, node_nums, W0, al0, ar0, b0, W1, al1, ar1, b1,
           W2, al2, ar2, b2):
    B, C, Nmax, Fin = x.shape
    Hh, Fh = al0.shape
    Fout = al2.shape[1]
    HF = Hh * Fh

    b0r = b0.reshape(1, HF)
    b1r = b1.reshape(1, HF)
    b2r = b2.reshape(1, Hh * Fout)

    def full(shape):
        return pl.BlockSpec(shape, lambda *_: (0,) * len(shape))

    grid_spec = pltpu.PrefetchScalarGridSpec(
        num_scalar_prefetch=1,
        grid=(1,),
        in_specs=[
            full((B, C, Nmax, Fin)),
            full((B, C, Nmax, Nmax)),
            full((Fin, HF)), full((Hh, Fh)), full((Hh, Fh)), full((1, HF)),
            full((HF, HF)), full((Hh, Fh)), full((Hh, Fh)), full((1, HF)),
            full((HF, Hh * Fout)), full((Hh, Fout)), full((Hh, Fout)),
            full((1, Hh * Fout)),
        ],
        out_specs=full((B, Nmax, Fout)),
    )

    out = pl.pallas_call(
        functools.partial(_gat_kernel, B=B, Nmax=Nmax, Fin=Fin,
                          Fh=Fh, Fout=Fout),
        grid_spec=grid_spec,
        out_shape=jax.ShapeDtypeStruct((B, Nmax, Fout), jnp.float32),
    )(node_nums.astype(jnp.int32), x, adj,
      W0, al0, ar0, b0r, W1, al1, ar1, b1r, W2, al2, ar2, b2r)
    return out
